# baseline (speedup 1.0000x reference)
import jax
import jax.numpy as jnp
from jax.experimental import pallas as pl
from jax.experimental.pallas import tpu as pltpu

_QB = 10000


def _postproc_body(obj_ref, verb_ref, sub_ref, objb_ref, scale_ref,
                   labels_ref, subo_ref, objo_ref, vs_ref, scores_ref):
    x = obj_ref[0]
    qb, c = x.shape
    col = jax.lax.broadcasted_iota(jnp.int32, x.shape, 1)
    xm = jnp.where(col < c - 1, x, -jnp.inf)
    m_obj = jnp.max(xm, axis=-1, keepdims=True)
    label = jnp.min(jnp.where(xm == m_obj, col, c), axis=-1, keepdims=True)
    score = 1.0 / jnp.sum(jnp.exp(x - m_obj), axis=-1, keepdims=True)

    vs_ref[0] = jax.nn.sigmoid(verb_ref[0]) * score

    scores_ref[0, 0] = score.reshape(8, qb // 8)
    lab = label.reshape(8, qb // 8)
    labels_ref[0, 0, 0] = jnp.zeros_like(lab)
    labels_ref[0, 1, 0] = lab

    scale = scale_ref[...]
    for src, dst in ((sub_ref, subo_ref), (objb_ref, objo_ref)):
        bx = src[...]
        cxy = bx[:, 0:2]
        half = bx[:, 2:4] * 0.5
        dst[...] = jnp.concatenate([cxy - half, cxy + half], axis=1) * scale


def kernel(pred_obj_logits, pred_verb_logits, pred_sub_boxes, pred_obj_boxes, target_sizes):
    B, Q, C = pred_obj_logits.shape
    V = pred_verb_logits.shape[-1]
    nq = Q // _QB
    qs = _QB // 8
    qbb = (-(-Q // (B * nq)) + 127) // 128 * 128

    img_h = target_sizes[:, 0].astype(jnp.float32)
    img_w = target_sizes[:, 1].astype(jnp.float32)
    scale = jnp.stack([img_w, img_h, img_w, img_h], axis=1).reshape(B, 4, 1)

    sub_t = jnp.transpose(pred_sub_boxes, (0, 2, 1))
    objb_t = jnp.transpose(pred_obj_boxes, (0, 2, 1))

    lab5, sub_o, obj_o, vs, sc4 = pl.pallas_call(
        _postproc_body,
        grid=(B, nq),
        in_specs=[
            pl.BlockSpec((1, _QB, C), lambda b, q: (b, q, 0)),
            pl.BlockSpec((1, _QB, V), lambda b, q: (b, q, 0)),
            pl.BlockSpec((B, 4, qbb), lambda b, q, n=nq: (0, 0, b * n + q)),
            pl.BlockSpec((B, 4, qbb), lambda b, q, n=nq: (0, 0, b * n + q)),
            pl.BlockSpec((B, 4, 1), lambda b, q: (0, 0, 0)),
        ],
        out_specs=[
            pl.BlockSpec((1, 2, 1, 8, qs), lambda b, q: (b, 0, q, 0, 0)),
            pl.BlockSpec((B, 4, qbb), lambda b, q, n=nq: (0, 0, b * n + q)),
            pl.BlockSpec((B, 4, qbb), lambda b, q, n=nq: (0, 0, b * n + q)),
            pl.BlockSpec((1, _QB, V), lambda b, q: (b, q, 0)),
            pl.BlockSpec((1, 1, 8, qs), lambda b, q: (b, q, 0, 0)),
        ],
        out_shape=[
            jax.ShapeDtypeStruct((B, 2, nq, 8, qs), jnp.int32),
            jax.ShapeDtypeStruct((B, 4, Q), jnp.float32),
            jax.ShapeDtypeStruct((B, 4, Q), jnp.float32),
            jax.ShapeDtypeStruct((B, Q, V), jnp.float32),
            jax.ShapeDtypeStruct((B, nq, 8, qs), jnp.float32),
        ],
        compiler_params=pltpu.CompilerParams(
            dimension_semantics=("parallel", "parallel")),
    )(pred_obj_logits, pred_verb_logits, sub_t, objb_t, scale)

    labels = lab5.reshape(B, 2 * Q)
    boxes = jnp.transpose(jnp.concatenate([sub_o, obj_o], axis=2), (0, 2, 1))
    obj_scores = sc4.reshape(B, Q)
    ids = jnp.arange(2 * Q)
    return (labels, boxes, vs, pred_verb_logits, ids[:Q], ids[Q:], obj_scores)


# speedup vs baseline: 1.0378x; 1.0378x over previous
import jax
import jax.numpy as jnp
from jax.experimental import pallas as pl
from jax.experimental.pallas import tpu as pltpu

_QB = 4000


def _postproc_body(obj_ref, verb_ref, sub_ref, objb_ref, scale_ref,
                   labels_ref, subo_ref, objo_ref, vs_ref, scores_ref):
    x = obj_ref[0]
    qb, c = x.shape
    col = jax.lax.broadcasted_iota(jnp.int32, x.shape, 1)
    xm = jnp.where(col < c - 1, x, -jnp.inf)
    m_obj = jnp.max(xm, axis=-1, keepdims=True)
    label = jnp.min(jnp.where(xm == m_obj, col, c), axis=-1, keepdims=True)
    score = 1.0 / jnp.sum(jnp.exp(x - m_obj), axis=-1, keepdims=True)

    vs_ref[0] = jax.nn.sigmoid(verb_ref[0]) * score

    scores_ref[0, 0] = score.reshape(8, qb // 8)
    lab = label.reshape(8, qb // 8)
    labels_ref[0, 0, 0] = jnp.zeros_like(lab)
    labels_ref[0, 1, 0] = lab

    scale = scale_ref[...]
    for src, dst in ((sub_ref, subo_ref), (objb_ref, objo_ref)):
        bx = src[...]
        cxy = bx[:, 0:2]
        half = bx[:, 2:4] * 0.5
        dst[...] = jnp.concatenate([cxy - half, cxy + half], axis=1) * scale


def kernel(pred_obj_logits, pred_verb_logits, pred_sub_boxes, pred_obj_boxes, target_sizes):
    B, Q, C = pred_obj_logits.shape
    V = pred_verb_logits.shape[-1]
    nq = Q // _QB
    qs = _QB // 8
    qbb = (-(-Q // (B * nq)) + 127) // 128 * 128

    img_h = target_sizes[:, 0].astype(jnp.float32)
    img_w = target_sizes[:, 1].astype(jnp.float32)
    scale = jnp.stack([img_w, img_h, img_w, img_h], axis=1).reshape(B, 4, 1)

    sub_t = jnp.transpose(pred_sub_boxes, (0, 2, 1))
    objb_t = jnp.transpose(pred_obj_boxes, (0, 2, 1))

    lab5, sub_o, obj_o, vs, sc4 = pl.pallas_call(
        _postproc_body,
        grid=(B, nq),
        in_specs=[
            pl.BlockSpec((1, _QB, C), lambda b, q: (b, q, 0)),
            pl.BlockSpec((1, _QB, V), lambda b, q: (b, q, 0)),
            pl.BlockSpec((B, 4, qbb), lambda b, q, n=nq: (0, 0, b * n + q)),
            pl.BlockSpec((B, 4, qbb), lambda b, q, n=nq: (0, 0, b * n + q)),
            pl.BlockSpec((B, 4, 1), lambda b, q: (0, 0, 0)),
        ],
        out_specs=[
            pl.BlockSpec((1, 2, 1, 8, qs), lambda b, q: (b, 0, q, 0, 0)),
            pl.BlockSpec((B, 4, qbb), lambda b, q, n=nq: (0, 0, b * n + q)),
            pl.BlockSpec((B, 4, qbb), lambda b, q, n=nq: (0, 0, b * n + q)),
            pl.BlockSpec((1, _QB, V), lambda b, q: (b, q, 0)),
            pl.BlockSpec((1, 1, 8, qs), lambda b, q: (b, q, 0, 0)),
        ],
        out_shape=[
            jax.ShapeDtypeStruct((B, 2, nq, 8, qs), jnp.int32),
            jax.ShapeDtypeStruct((B, 4, Q), jnp.float32),
            jax.ShapeDtypeStruct((B, 4, Q), jnp.float32),
            jax.ShapeDtypeStruct((B, Q, V), jnp.float32),
            jax.ShapeDtypeStruct((B, nq, 8, qs), jnp.float32),
        ],
        compiler_params=pltpu.CompilerParams(
            dimension_semantics=("parallel", "parallel")),
    )(pred_obj_logits, pred_verb_logits, sub_t, objb_t, scale)

    labels = lab5.reshape(B, 2 * Q)
    boxes = jnp.transpose(jnp.concatenate([sub_o, obj_o], axis=2), (0, 2, 1))
    obj_scores = sc4.reshape(B, Q)
    ids = jnp.arange(2 * Q)
    return (labels, boxes, vs, pred_verb_logits, ids[:Q], ids[Q:], obj_scores)
